# baseline (speedup 1.0000x reference)
import functools

import jax
import jax.numpy as jnp
from jax import lax
from jax.experimental import pallas as pl
from jax.experimental.pallas import tpu as pltpu
from jax.experimental.pallas import tpu_sc as plsc

_NUM_CORES = 2
_NUM_SUBCORES = 16
_CHUNK = 512


@functools.cache
def _make_gather(B, D):
    nw = _NUM_CORES * _NUM_SUBCORES
    assert B % (8 * nw) == 0
    b_per_w = B // nw
    assert b_per_w % _CHUNK == 0
    n_chunks = b_per_w // _CHUNK
    mesh = plsc.VectorSubcoreMesh(core_axis_name="c", subcore_axis_name="s")

    @functools.partial(
        pl.kernel,
        out_type=jax.ShapeDtypeStruct((B, D), jnp.float32),
        mesh=mesh,
        scratch_types=[
            pltpu.VMEM((_CHUNK,), jnp.int32),
            pltpu.VMEM((_CHUNK, D), jnp.float32),
            pltpu.SemaphoreType.DMA,
        ],
        compiler_params=pltpu.CompilerParams(use_tc_tiling_on_sc=False),
    )
    def gather_kernel(idx_hbm, table_hbm, out_hbm, idx_v, rows_v, sem):
        wid = lax.axis_index("s") * _NUM_CORES + lax.axis_index("c")
        base = wid * b_per_w

        def body(i, carry):
            off = pl.multiple_of(base + i * _CHUNK, _CHUNK)
            pltpu.sync_copy(idx_hbm.at[pl.ds(off, _CHUNK)], idx_v)
            pltpu.async_copy(table_hbm.at[idx_v], rows_v, sem).wait()
            pltpu.sync_copy(rows_v, out_hbm.at[pl.ds(off, _CHUNK)])
            return carry

        lax.fori_loop(0, n_chunks, body, 0)

    return gather_kernel


def kernel(token_ids, weight):
    bsz, seq = token_ids.shape
    _, d = weight.shape
    flat = token_ids.reshape(bsz * seq).astype(jnp.int32)
    out = _make_gather(bsz * seq, d)(flat, weight)
    return out.reshape(bsz, seq, d)


# speedup vs baseline: 1.0406x; 1.0406x over previous
import functools

import jax
import jax.numpy as jnp
from jax import lax
from jax.experimental import pallas as pl
from jax.experimental.pallas import tpu as pltpu
from jax.experimental.pallas import tpu_sc as plsc

_NUM_CORES = 2
_NUM_SUBCORES = 16
_CHUNK = 512


@functools.cache
def _make_gather(B, D):
    nw = _NUM_CORES * _NUM_SUBCORES
    assert B % (8 * nw) == 0
    b_per_w = B // nw
    assert b_per_w % (2 * _CHUNK) == 0
    n_loops = b_per_w // (2 * _CHUNK)
    mesh = plsc.VectorSubcoreMesh(core_axis_name="c", subcore_axis_name="s")

    @functools.partial(
        pl.kernel,
        out_type=jax.ShapeDtypeStruct((B, D), jnp.float32),
        mesh=mesh,
        scratch_types=[
            pltpu.VMEM((b_per_w,), jnp.int32),
            pltpu.VMEM((2, _CHUNK, D), jnp.float32),
            pltpu.SemaphoreType.DMA((2,)),
        ],
        compiler_params=pltpu.CompilerParams(use_tc_tiling_on_sc=False),
    )
    def gather_kernel(idx_hbm, table_hbm, out_hbm, idx_v, rows_v, gsem):
        wid = lax.axis_index("s") * _NUM_CORES + lax.axis_index("c")
        base = pl.multiple_of(wid * b_per_w, 8)

        pltpu.sync_copy(idx_hbm.at[pl.ds(base, b_per_w)], idx_v)

        def start_gather(c, b):
            loc = pl.multiple_of(c * _CHUNK, 8)
            pltpu.async_copy(
                table_hbm.at[idx_v.at[pl.ds(loc, _CHUNK)]], rows_v.at[b],
                gsem.at[b])

        def wait_gather(c, b):
            loc = pl.multiple_of(c * _CHUNK, 8)
            pltpu.make_async_copy(
                table_hbm.at[idx_v.at[pl.ds(loc, _CHUNK)]], rows_v.at[b],
                gsem.at[b]).wait()

        start_gather(0, 0)

        def body(j, carry):
            for b in (0, 1):
                c = 2 * j + b
                wait_gather(c, b)
                if b == 0:
                    start_gather(c + 1, 1)
                else:

                    @pl.when(j < n_loops - 1)
                    def _():
                        start_gather(c + 1, 0)

                glob = pl.multiple_of(base + c * _CHUNK, 8)
                pltpu.sync_copy(rows_v.at[b], out_hbm.at[pl.ds(glob, _CHUNK)])
            return carry

        lax.fori_loop(0, n_loops, body, 0)

    return gather_kernel


def kernel(token_ids, weight):
    bsz, seq = token_ids.shape
    _, d = weight.shape
    flat = token_ids.reshape(bsz * seq).astype(jnp.int32)
    out = _make_gather(bsz * seq, d)(flat, weight)
    return out.reshape(bsz, seq, d)
